# baseline (speedup 1.0000x reference)
import functools

import jax
import jax.numpy as jnp
from jax import lax
from jax.experimental import pallas as pl
from jax.experimental.pallas import tpu as pltpu
from jax.experimental.pallas import tpu_sc as plsc

_BATCH = 1024
_TOTAL = 100000
_CONN = 128
_L = 16
_NW = 32
_BPW = _BATCH // _NW


@functools.lru_cache(maxsize=1)
def _build_sc_call():
    mesh = plsc.VectorSubcoreMesh(core_axis_name="c", subcore_axis_name="s")

    @functools.partial(
        pl.kernel,
        mesh=mesh,
        compiler_params=pltpu.CompilerParams(needs_layout_passes=False),
        out_type=jax.ShapeDtypeStruct((_BATCH,), jnp.float32),
        scratch_types=[
            pltpu.VMEM((_CONN,), jnp.int32),
            pltpu.VMEM((_CONN,), jnp.float32),
            pltpu.VMEM((_L,), jnp.float32),
            pltpu.VMEM((_BPW, _CONN), jnp.int32),
            pltpu.VMEM((_BPW * _CONN,), jnp.float32),
            pltpu.VMEM((_BPW,), jnp.float32),
            pltpu.SemaphoreType.DMA,
        ],
    )
    def sc_call(act_hbm, w_hbm, params_hbm, idx_hbm, out_hbm,
                idx_v, w_v, params_v, rows_v, buf_v, out_v, sem):
        wid = lax.axis_index("s") * 2 + lax.axis_index("c")
        base = wid * _BPW
        pltpu.sync_copy(idx_hbm, idx_v)
        pltpu.sync_copy(w_hbm, w_v)
        pltpu.sync_copy(params_hbm, params_v)
        for g in range(_CONN // _L):
            col = idx_v[pl.ds(g * _L, _L)]
            for b in range(_BPW):
                rows_v[b, pl.ds(g * _L, _L)] = col + (base + b) * _TOTAL
        copies = [pltpu.async_copy(act_hbm.at[rows_v.at[b]],
                                   buf_v.at[pl.ds(b * _CONN, _CONN)], sem)
                  for b in range(_BPW)]
        for c in copies:
            c.wait()
        pv = params_v[pl.ds(0, _L)]
        sens = pv[0]
        thr = pv[1]
        wgs = [w_v[pl.ds(g * _L, _L)] for g in range(_CONN // _L)]
        for grp in range(_BPW // _L):
            sums = []
            for bb in range(_L):
                b = grp * _L + bb
                acc = buf_v[pl.ds(b * _CONN, _L)] * wgs[0]
                for g in range(1, _CONN // _L):
                    acc = acc + buf_v[pl.ds(b * _CONN + g * _L, _L)] * wgs[g]
                sums.append(jnp.sum(acc))
            lanes = lax.iota(jnp.int32, _L)
            z = jnp.zeros((_L,), jnp.float32)
            for bb, s in enumerate(sums):
                z = jnp.where(lanes == bb, s, z)
            z = z * sens - thr
            out_v[pl.ds(grp * _L, _L)] = 1.0 / (1.0 + jnp.exp(-z))
        pltpu.sync_copy(out_v, out_hbm.at[pl.ds(base, _BPW)])

    return sc_call


def kernel(x, all_activations, connection_weights, sensitivity, threshold, connection_indices):
    del x
    act_flat = all_activations.reshape(-1)
    params = jnp.concatenate(
        [sensitivity, threshold, jnp.zeros((_L - 2,), jnp.float32)])
    return _build_sc_call()(act_flat, connection_weights, params, connection_indices)


# speedup vs baseline: 1.0026x; 1.0026x over previous
import functools

import jax
import jax.numpy as jnp
from jax import lax
from jax.experimental import pallas as pl
from jax.experimental.pallas import tpu as pltpu
from jax.experimental.pallas import tpu_sc as plsc

_BATCH = 1024
_TOTAL = 100000
_CONN = 128
_L = 16
_NW = 32
_BPW = _BATCH // _NW


@functools.lru_cache(maxsize=1)
def _build_sc_call():
    mesh = plsc.VectorSubcoreMesh(core_axis_name="c", subcore_axis_name="s")

    @functools.partial(
        pl.kernel,
        mesh=mesh,
        compiler_params=pltpu.CompilerParams(needs_layout_passes=False),
        out_type=jax.ShapeDtypeStruct((_BATCH,), jnp.float32),
        scratch_types=[
            pltpu.VMEM((_CONN,), jnp.int32),
            pltpu.VMEM((_CONN,), jnp.float32),
            pltpu.VMEM((_L,), jnp.float32),
            pltpu.VMEM((_BPW * _CONN,), jnp.int32),
            pltpu.VMEM((_BPW * _CONN,), jnp.float32),
            pltpu.VMEM((_BPW,), jnp.float32),
            pltpu.SemaphoreType.DMA,
        ],
    )
    def sc_call(act_hbm, w_hbm, params_hbm, idx_hbm, out_hbm,
                idx_v, w_v, params_v, rows_v, buf_v, out_v, sem):
        wid = lax.axis_index("s") * 2 + lax.axis_index("c")
        base = wid * _BPW
        pltpu.sync_copy(idx_hbm, idx_v)
        pltpu.sync_copy(w_hbm, w_v)
        pltpu.sync_copy(params_hbm, params_v)
        for g in range(_CONN // _L):
            col = idx_v[pl.ds(g * _L, _L)]
            for b in range(_BPW):
                rows_v[pl.ds(b * _CONN + g * _L, _L)] = col + (base + b) * _TOTAL
        pltpu.async_copy(act_hbm.at[rows_v], buf_v, sem).wait()
        pv = params_v[pl.ds(0, _L)]
        sens = pv[0]
        thr = pv[1]
        wgs = [w_v[pl.ds(g * _L, _L)] for g in range(_CONN // _L)]
        for grp in range(_BPW // _L):
            sums = []
            for bb in range(_L):
                b = grp * _L + bb
                acc = buf_v[pl.ds(b * _CONN, _L)] * wgs[0]
                for g in range(1, _CONN // _L):
                    acc = acc + buf_v[pl.ds(b * _CONN + g * _L, _L)] * wgs[g]
                sums.append(jnp.sum(acc))
            lanes = lax.iota(jnp.int32, _L)
            z = jnp.zeros((_L,), jnp.float32)
            for bb, s in enumerate(sums):
                z = jnp.where(lanes == bb, s, z)
            z = z * sens - thr
            out_v[pl.ds(grp * _L, _L)] = 1.0 / (1.0 + jnp.exp(-z))
        pltpu.sync_copy(out_v, out_hbm.at[pl.ds(base, _BPW)])

    return sc_call


def kernel(x, all_activations, connection_weights, sensitivity, threshold, connection_indices):
    del x
    act_flat = all_activations.reshape(-1)
    params = jnp.concatenate(
        [sensitivity, threshold, jnp.zeros((_L - 2,), jnp.float32)])
    return _build_sc_call()(act_flat, connection_weights, params, connection_indices)


# speedup vs baseline: 2.3212x; 2.3153x over previous
import jax
import jax.numpy as jnp
from jax import lax
from jax.experimental import pallas as pl
from jax.experimental.pallas import tpu as pltpu

_BATCH = 1024
_TOTAL = 100000
_CONN = 128
_LANES = 128
_NBUF = 8


def _strip_copy(a_ref, idx_ref, bufs, sems, j):
    col = idx_ref[j]
    off = pl.multiple_of((col // _LANES) * _LANES, _LANES)
    return pltpu.make_async_copy(
        a_ref.at[:, pl.ds(off, _LANES)],
        bufs.at[j % _NBUF],
        sems.at[j % _NBUF],
    )


def _body(idx_ref, a_ref, w_ref, sens_ref, thr_ref, o_ref, bufs, sems):
    for j in range(_NBUF):
        _strip_copy(a_ref, idx_ref, bufs, sems, j).start()
    lane = lax.broadcasted_iota(jnp.int32, (1, _LANES), 1)
    acc = jnp.zeros((_BATCH, _LANES), jnp.float32)
    for j in range(_CONN):
        _strip_copy(a_ref, idx_ref, bufs, sems, j).wait()
        sel = jnp.where(lane == idx_ref[j] % _LANES, w_ref[j], 0.0)
        acc = acc + bufs[j % _NBUF] * sel
        if j + _NBUF < _CONN:
            _strip_copy(a_ref, idx_ref, bufs, sems, j + _NBUF).start()
    z = jnp.sum(acc, axis=1)
    z = z * sens_ref[0] - thr_ref[0]
    o_ref[...] = 1.0 / (1.0 + jnp.exp(-z))


def kernel(x, all_activations, connection_weights, sensitivity, threshold,
           connection_indices):
    del x
    return pl.pallas_call(
        _body,
        grid_spec=pltpu.PrefetchScalarGridSpec(
            num_scalar_prefetch=1,
            in_specs=[
                pl.BlockSpec(memory_space=pl.ANY),
                pl.BlockSpec(memory_space=pltpu.SMEM),
                pl.BlockSpec(memory_space=pltpu.SMEM),
                pl.BlockSpec(memory_space=pltpu.SMEM),
            ],
            out_specs=pl.BlockSpec(memory_space=pltpu.VMEM),
            scratch_shapes=[
                pltpu.VMEM((_NBUF, _BATCH, _LANES), jnp.float32),
                pltpu.SemaphoreType.DMA((_NBUF,)),
            ],
        ),
        out_shape=jax.ShapeDtypeStruct((_BATCH,), jnp.float32),
    )(connection_indices, all_activations, connection_weights,
      sensitivity, threshold)


# speedup vs baseline: 2.3500x; 1.0124x over previous
import jax
import jax.numpy as jnp
from jax import lax
from jax.experimental import pallas as pl
from jax.experimental.pallas import tpu as pltpu

_BATCH = 1024
_TOTAL = 100000
_CONN = 128
_LANES = 128
_NBUF = 16
_SPLIT = 4
_RB = _BATCH // _SPLIT


def _strip_copies(a_ref, idx_ref, bufs, sems, j):
    col = idx_ref[j]
    off = pl.multiple_of((col // _LANES) * _LANES, _LANES)
    return [
        pltpu.make_async_copy(
            a_ref.at[pl.ds(r * _RB, _RB), pl.ds(off, _LANES)],
            bufs.at[j % _NBUF, pl.ds(r * _RB, _RB)],
            sems.at[j % _NBUF],
        )
        for r in range(_SPLIT)
    ]


def _body(idx_ref, a_ref, w_ref, sens_ref, thr_ref, o_ref, bufs, sems):
    for j in range(_NBUF):
        for c in _strip_copies(a_ref, idx_ref, bufs, sems, j):
            c.start()
    lane = lax.broadcasted_iota(jnp.int32, (1, _LANES), 1)
    acc = jnp.zeros((_BATCH, _LANES), jnp.float32)
    for j in range(_CONN):
        for c in _strip_copies(a_ref, idx_ref, bufs, sems, j):
            c.wait()
        sel = jnp.where(lane == idx_ref[j] % _LANES, w_ref[j], 0.0)
        acc = acc + bufs[j % _NBUF] * sel
        if j + _NBUF < _CONN:
            for c in _strip_copies(a_ref, idx_ref, bufs, sems, j + _NBUF):
                c.start()
    z = jnp.sum(acc, axis=1)
    z = z * sens_ref[0] - thr_ref[0]
    o_ref[...] = 1.0 / (1.0 + jnp.exp(-z))


def kernel(x, all_activations, connection_weights, sensitivity, threshold,
           connection_indices):
    del x
    return pl.pallas_call(
        _body,
        grid_spec=pltpu.PrefetchScalarGridSpec(
            num_scalar_prefetch=1,
            in_specs=[
                pl.BlockSpec(memory_space=pl.ANY),
                pl.BlockSpec(memory_space=pltpu.SMEM),
                pl.BlockSpec(memory_space=pltpu.SMEM),
                pl.BlockSpec(memory_space=pltpu.SMEM),
            ],
            out_specs=pl.BlockSpec(memory_space=pltpu.VMEM),
            scratch_shapes=[
                pltpu.VMEM((_NBUF, _BATCH, _LANES), jnp.float32),
                pltpu.SemaphoreType.DMA((_NBUF,)),
            ],
        ),
        out_shape=jax.ShapeDtypeStruct((_BATCH,), jnp.float32),
    )(connection_indices, all_activations, connection_weights,
      sensitivity, threshold)
